# baseline (speedup 1.0000x reference)
import functools

import jax
import jax.numpy as jnp
from jax import lax
from jax.experimental import pallas as pl
from jax.experimental.pallas import tpu as pltpu
from jax.experimental.pallas import tpu_sc as plsc

NC = 2
NS = 16
NW = NC * NS

LANES = 16
ISLAB = 128
JQ = 50


def _pair_table_body(e_ref, w_ref, b_ref, o_ref):
    h = (
        jnp.dot(e_ref[...], w_ref[...], preferred_element_type=jnp.float32)
        + b_ref[...]
    )
    v, d = h.shape
    a = jnp.broadcast_to(h[:, None, :], (v, v, d))
    bb = jnp.broadcast_to(h[None, :, :], (v, v, d))
    o_ref[...] = jnp.concatenate([a, bb], axis=-1)


def _make_sc_gather(bs, sl, v, d):
    d2 = 2 * d
    assert d2 == LANES and sl % (2 * JQ) == 0 and bs % (NW * ISLAB) == 0
    rows_i = bs // NW
    nslab_i = rows_i // ISLAB
    nslab_j = sl // JQ
    slab_tok = ISLAB * sl
    gp = JQ // 2
    nsub = ISLAB // LANES
    ngrp = gp * nsub
    tsz = v * v * d2

    mesh = plsc.VectorSubcoreMesh(core_axis_name="c", subcore_axis_name="s")

    @functools.partial(
        pl.kernel,
        out_type=jax.ShapeDtypeStruct((sl, d, bs), jnp.float32),
        mesh=mesh,
        scratch_types=[
            pltpu.VMEM((sl, ISLAB), jnp.int32),
            pltpu.VMEM((tsz,), jnp.float32),
            pltpu.VMEM((JQ, d, ISLAB), jnp.float32),
        ],
        compiler_params=pltpu.CompilerParams(
            use_tc_tiling_on_sc=True, needs_layout_passes=False
        ),
    )
    def sc_gather(x_hbm, t2_hbm, out_hbm, xbig, t2t, rows):
        wid = lax.axis_index("s") * NC + lax.axis_index("c")
        pltpu.sync_copy(t2_hbm, t2t)

        iota = lax.iota(jnp.int32, LANES)
        zero = iota * 0
        kvs = [zero + k for k in range(d)]
        wi0 = wid * rows_i

        def islab(si, carry):
            pltpu.sync_copy(
                x_hbm.at[slice(None), pl.ds(wi0 + si * ISLAB, ISLAB)], xbig
            )

            def jslab(jq, carry2):
                @plsc.parallel_loop(0, ngrp, unroll=4)
                def group(q):
                    g = q >> 3
                    sub = q & (nsub - 1)
                    ivec = sub * LANES + iota
                    jv0 = zero + (jq * JQ + 2 * g)
                    jv1 = jv0 + 1
                    ev = plsc.load_gather(xbig, [jv0, ivec])
                    od = plsc.load_gather(xbig, [jv1, ivec])
                    p16 = (ev * v + od) * d2
                    rj0 = zero + 2 * g
                    rj1 = rj0 + 1
                    for c in range(d2):
                        vals = plsc.load_gather(t2t, [p16 + c])
                        plsc.store_scatter(
                            rows,
                            [rj0 if c < d else rj1, kvs[c % d], ivec],
                            vals,
                        )

                pltpu.sync_copy(
                    rows,
                    out_hbm.at[
                        pl.ds(jq * JQ, JQ),
                        slice(None),
                        pl.ds(wi0 + si * ISLAB, ISLAB),
                    ],
                )
                return carry2

            lax.fori_loop(0, nslab_j, jslab, 0)
            return carry

        lax.fori_loop(0, nslab_i, islab, 0)

    return sc_gather


def kernel(x, embed_table, W, b):
    bs, sl = x.shape
    v = embed_table.shape[0]
    d = W.shape[1]
    t2 = pl.pallas_call(
        _pair_table_body,
        out_shape=jax.ShapeDtypeStruct((v, v, 2 * d), jnp.float32),
    )(embed_table, W, b.reshape(1, d))
    xt = jnp.transpose(x)
    jki = _make_sc_gather(bs, sl, v, d)(xt, t2.reshape(v * v * 2 * d))
    return jnp.transpose(jki, (2, 0, 1))
